# baseline (speedup 1.0000x reference)
import jax
import jax.numpy as jnp
from jax import lax
from jax.experimental import pallas as pl
from jax.experimental.compute_on import compute_on

N = 10000
E_EDGES = 160000
HID = 256

G = 128
NBK = 79
CAP = 2560
E_PAD = NBK * CAP
N_PAD = NBK * G


def _linear_kernel(x_ref, w_ref, b_ref, o_ref):
    o_ref[...] = (
        jnp.dot(x_ref[...], w_ref[...], preferred_element_type=jnp.float32)
        + b_ref[...]
    )


def _linear(x, w, b, block_m=512):
    m, k = x.shape
    f = w.shape[1]
    grid = (pl.cdiv(m, block_m),)
    return pl.pallas_call(
        _linear_kernel,
        grid=grid,
        in_specs=[
            pl.BlockSpec((block_m, k), lambda i: (i, 0)),
            pl.BlockSpec((k, f), lambda i: (0, 0)),
            pl.BlockSpec((f,), lambda i: (0,)),
        ],
        out_specs=pl.BlockSpec((block_m, f), lambda i: (i, 0)),
        out_shape=jax.ShapeDtypeStruct((m, f), jnp.float32),
    )(x, w, b)


def _edge_kernel(
    eprev_ref, enprev_ref, a_ref, c_ref, g1_ref, eh_ref, dl_ref, sn_ref,
    cw_ref, cb_ref, el_ref, en_ref, nd_ref, s_ref, *, first, last
):
    if first:
        e_l = eprev_ref[...] * a_ref[...] + c_ref[...]
    else:
        e_l = eprev_ref[...] + jnp.maximum(enprev_ref[...] * a_ref[...] + c_ref[...], 0.0)
    if not last:
        el_ref[...] = e_l
    ce = jnp.dot(e_l, cw_ref[...], preferred_element_type=jnp.float32) + cb_ref[...]
    dl = dl_ref[0, 0, :]
    iota = lax.broadcasted_iota(jnp.int32, (CAP, G), 1)
    oh = (dl[:, None] == iota).astype(jnp.float32)
    ehd = jnp.dot(oh, eh_ref[...], preferred_element_type=jnp.float32)
    g1 = g1_ref[...].astype(jnp.float32)
    e_pre = g1[:, :HID] + ehd + ce
    sigma = jax.nn.sigmoid(e_pre)
    ps = jnp.concatenate([sigma * g1[:, HID:], sigma], axis=1)
    nd_ref[...] = lax.dot_general(
        oh, ps, (((0,), (0,)), ((), ())), preferred_element_type=jnp.float32
    )
    en = e_pre * sn_ref[...]
    if not last:
        en_ref[...] = en
    bk = pl.program_id(0)

    @pl.when(bk == 0)
    def _():
        s_ref[...] = jnp.zeros_like(s_ref)

    s1 = jnp.sum(en, axis=0, keepdims=True)
    s2 = jnp.sum(en * en, axis=0, keepdims=True)
    s_ref[0:1, :] = s_ref[0:1, :] + jnp.concatenate([s1, s2], axis=1)


def _edge_stage(e_prev, en_prev, a, c, g1, eh, dst_local, sn, cw, cb, first, last):
    e_cols = 1 if first else HID
    kern = lambda *refs: _edge_kernel(*refs, first=first, last=last)
    out_shapes = [
        jax.ShapeDtypeStruct((E_PAD, HID), jnp.float32),
        jax.ShapeDtypeStruct((E_PAD, HID), jnp.float32),
        jax.ShapeDtypeStruct((N_PAD, 2 * HID), jnp.float32),
        jax.ShapeDtypeStruct((1, 2 * HID), jnp.float32),
    ]
    out_specs = [
        pl.BlockSpec((CAP, HID), lambda g: (g, 0)),
        pl.BlockSpec((CAP, HID), lambda g: (g, 0)),
        pl.BlockSpec((G, 2 * HID), lambda g: (g, 0)),
        pl.BlockSpec((1, 2 * HID), lambda g: (0, 0)),
    ]
    in_specs = [
        pl.BlockSpec((CAP, e_cols), lambda g: (g, 0)),
        pl.BlockSpec((CAP, e_cols), lambda g: (g, 0)),
        pl.BlockSpec((HID,), lambda g: (0,)),
        pl.BlockSpec((HID,), lambda g: (0,)),
        pl.BlockSpec((CAP, 2 * HID), lambda g: (g, 0)),
        pl.BlockSpec((G, HID), lambda g: (g, 0)),
        pl.BlockSpec((1, 1, CAP), lambda g: (g, 0, 0)),
        pl.BlockSpec((CAP, 1), lambda g: (g, 0)),
        pl.BlockSpec((HID, HID), lambda g: (0, 0)),
        pl.BlockSpec((HID,), lambda g: (0,)),
    ]
    return pl.pallas_call(
        kern,
        grid=(NBK,),
        in_specs=in_specs,
        out_specs=out_specs,
        out_shape=out_shapes,
    )(e_prev, en_prev, a, c, g1, eh, dst_local.reshape(NBK, 1, CAP), sn, cw, cb)


@compute_on("tpu_sparsecore")
@jax.jit
def _take_sc(table, idx):
    return jnp.take(table, idx, axis=0, mode="clip")


def _bn_nodes(x, g, b):
    m = x.mean(axis=0)
    v = x.var(axis=0)
    return (x - m) / jnp.sqrt(v + 1e-5) * g + b


def kernel(h, e, edge_index, snorm_n, snorm_e, label, delta, params):
    src = edge_index[0]
    dst = edge_index[1]

    bucket = dst // G
    onehot = (bucket[:, None] == jnp.arange(NBK)[None, :]).astype(jnp.int32)
    ranks = jnp.cumsum(onehot, axis=0) - onehot
    rank = jnp.take_along_axis(ranks, bucket[:, None], axis=1)[:, 0]
    slot = bucket * CAP + rank

    dst_local = jnp.full((E_PAD,), -1, jnp.int32).at[slot].set(dst - bucket * G)
    src_pad = jnp.zeros((E_PAD,), jnp.int32).at[slot].set(src)
    e0_pad = jnp.zeros((E_PAD, 1), jnp.float32).at[slot].set(e)
    sn_pad = jnp.zeros((E_PAD, 1), jnp.float32).at[slot].set(snorm_e)

    hv = params["emb_h"][h]

    e_prev = e0_pad
    en_prev = e0_pad
    a_vec = params["emb_e"]["w"][0]
    c_vec = params["emb_e"]["b"]
    nlayers = len(params["layers"])
    for li, p in enumerate(params["layers"]):
        first = li == 0
        last = li == nlayers - 1
        w_abde = jnp.concatenate(
            [p["A"]["w"], p["D"]["w"], p["B"]["w"], p["E"]["w"]], axis=1
        )
        b_abde = jnp.concatenate(
            [p["A"]["b"], p["D"]["b"], p["B"]["b"], p["E"]["b"]], axis=0
        )
        abde = _linear(hv, w_abde, b_abde, block_m=400)
        Ah = abde[:, :HID]
        DB = abde[:, HID : 3 * HID]
        Eh = abde[:, 3 * HID :]
        Eh_pad = jnp.pad(Eh, ((0, N_PAD - N), (0, 0)))
        g1 = _take_sc(DB.astype(jnp.bfloat16), src_pad)
        e_l, en_l, num_den, s12 = _edge_stage(
            e_prev, en_prev, a_vec, c_vec, g1, Eh_pad, dst_local, sn_pad,
            p["C"]["w"], p["C"]["b"], first, last,
        )
        num = num_den[:N, :HID]
        den = num_den[:N, HID:]
        h_new = Ah + num / (den + 1e-6)
        h_new = h_new * snorm_n
        h_new = _bn_nodes(h_new, p["bn_h_g"], p["bn_h_b"])
        hv = hv + jax.nn.relu(h_new)
        m_e = s12[0, :HID] / E_EDGES
        v_e = s12[0, HID:] / E_EDGES - m_e * m_e
        a_vec = p["bn_e_g"] / jnp.sqrt(v_e + 1e-5)
        c_vec = p["bn_e_b"] - m_e * a_vec
        e_prev, en_prev = e_l, en_l

    x = hv
    nmlp = len(params["mlp"])
    for i, lin in enumerate(params["mlp"]):
        x = _linear(x, lin["w"], lin["b"], block_m=2000)
        if i < nmlp - 1:
            x = jax.nn.relu(x)
    p_out = x
    hc = jnp.concatenate([hv, label], axis=1)
    w = jax.nn.sigmoid(_linear(hc, params["mlp2"]["w"], params["mlp2"]["b"], block_m=2000))
    w = jnp.tile(w, (1, label.shape[1]))
    w = jnp.clip(w, 0.0, jnp.asarray(delta, dtype=jnp.float32))
    ones = jnp.ones_like(label)
    max_entropy = jnp.full_like(label, 1.0 / label.shape[1])
    g_hat = (ones - w) * label + w * max_entropy
    return p_out, g_hat


# speedup vs baseline: 2.1871x; 2.1871x over previous
import jax
import jax.numpy as jnp
from jax import lax
from jax.experimental import pallas as pl
from jax.experimental.compute_on import compute_on

N = 10000
E_EDGES = 160000
HID = 256

G = 128
NBK = 79
CAP = 2560
E_PAD = NBK * CAP
N_PAD = NBK * G


def _linear_kernel(x_ref, w_ref, b_ref, o_ref):
    o_ref[...] = (
        jnp.dot(x_ref[...], w_ref[...], preferred_element_type=jnp.float32)
        + b_ref[...]
    )


def _linear(x, w, b, block_m=512):
    m, k = x.shape
    f = w.shape[1]
    grid = (pl.cdiv(m, block_m),)
    return pl.pallas_call(
        _linear_kernel,
        grid=grid,
        in_specs=[
            pl.BlockSpec((block_m, k), lambda i: (i, 0)),
            pl.BlockSpec((k, f), lambda i: (0, 0)),
            pl.BlockSpec((f,), lambda i: (0,)),
        ],
        out_specs=pl.BlockSpec((block_m, f), lambda i: (i, 0)),
        out_shape=jax.ShapeDtypeStruct((m, f), jnp.float32),
    )(x, w, b)


def _edge_kernel(
    eprev_ref, enprev_ref, a_ref, c_ref, g1_ref, wdb_ref, bdb_ref, eh_ref,
    dl_ref, sn_ref, cw_ref, cb_ref, el_ref, en_ref, nd_ref, s_ref, *, first, last
):
    if first:
        e_l = eprev_ref[...] * a_ref[...] + c_ref[...]
    else:
        e_l = eprev_ref[...].astype(jnp.float32) + jnp.maximum(
            enprev_ref[...].astype(jnp.float32) * a_ref[...] + c_ref[...], 0.0
        )
    if not last:
        el_ref[...] = e_l.astype(el_ref.dtype)
    ce = jnp.dot(e_l, cw_ref[...], preferred_element_type=jnp.float32) + cb_ref[...]
    dl = dl_ref[0, 0, :]
    iota = lax.broadcasted_iota(jnp.int32, (CAP, G), 1)
    oh = (dl[:, None] == iota).astype(jnp.float32)
    ehd = jnp.dot(oh, eh_ref[...], preferred_element_type=jnp.float32)
    db_s = (
        jnp.dot(g1_ref[...], wdb_ref[...], preferred_element_type=jnp.float32)
        + bdb_ref[...]
    )
    e_pre = db_s[:, :HID] + ehd + ce
    sigma = jax.nn.sigmoid(e_pre)
    ps = jnp.concatenate([sigma * db_s[:, HID:], sigma], axis=1)
    nd_ref[...] = lax.dot_general(
        oh, ps, (((0,), (0,)), ((), ())), preferred_element_type=jnp.float32
    )
    en = e_pre * sn_ref[...]
    if not last:
        en_ref[...] = en.astype(en_ref.dtype)
    bk = pl.program_id(0)

    @pl.when(bk == 0)
    def _():
        s_ref[...] = jnp.zeros_like(s_ref)

    s1 = jnp.sum(en, axis=0, keepdims=True)
    s2 = jnp.sum(en * en, axis=0, keepdims=True)
    s_ref[0:1, :] = s_ref[0:1, :] + jnp.concatenate([s1, s2], axis=1)


def _edge_stage(e_prev, en_prev, a, c, g1, wdb, bdb, eh, dst_local, sn, cw, cb,
                first, last):
    e_cols = 1 if first else HID
    kern = lambda *refs: _edge_kernel(*refs, first=first, last=last)
    out_shapes = [
        jax.ShapeDtypeStruct((E_PAD, HID), jnp.bfloat16),
        jax.ShapeDtypeStruct((E_PAD, HID), jnp.bfloat16),
        jax.ShapeDtypeStruct((N_PAD, 2 * HID), jnp.float32),
        jax.ShapeDtypeStruct((1, 2 * HID), jnp.float32),
    ]
    out_specs = [
        pl.BlockSpec((CAP, HID), lambda g: (g, 0)),
        pl.BlockSpec((CAP, HID), lambda g: (g, 0)),
        pl.BlockSpec((G, 2 * HID), lambda g: (g, 0)),
        pl.BlockSpec((1, 2 * HID), lambda g: (0, 0)),
    ]
    in_specs = [
        pl.BlockSpec((CAP, e_cols), lambda g: (g, 0)),
        pl.BlockSpec((CAP, e_cols), lambda g: (g, 0)),
        pl.BlockSpec((HID,), lambda g: (0,)),
        pl.BlockSpec((HID,), lambda g: (0,)),
        pl.BlockSpec((CAP, HID), lambda g: (g, 0)),
        pl.BlockSpec((HID, 2 * HID), lambda g: (0, 0)),
        pl.BlockSpec((2 * HID,), lambda g: (0,)),
        pl.BlockSpec((G, HID), lambda g: (g, 0)),
        pl.BlockSpec((1, 1, CAP), lambda g: (g, 0, 0)),
        pl.BlockSpec((CAP, 1), lambda g: (g, 0)),
        pl.BlockSpec((HID, HID), lambda g: (0, 0)),
        pl.BlockSpec((HID,), lambda g: (0,)),
    ]
    return pl.pallas_call(
        kern,
        grid=(NBK,),
        in_specs=in_specs,
        out_specs=out_specs,
        out_shape=out_shapes,
    )(e_prev, en_prev, a, c, g1, wdb, bdb, eh,
      dst_local.reshape(NBK, 1, CAP), sn, cw, cb)


@compute_on("tpu_sparsecore")
@jax.jit
def _take_sc(table, idx):
    return jnp.take(table, idx, axis=0, mode="clip")


def _bn_nodes(x, g, b):
    m = x.mean(axis=0)
    v = x.var(axis=0)
    return (x - m) / jnp.sqrt(v + 1e-5) * g + b


def kernel(h, e, edge_index, snorm_n, snorm_e, label, delta, params):
    src = edge_index[0]
    dst = edge_index[1]

    bucket = dst // G
    onehot = (bucket[:, None] == jnp.arange(NBK)[None, :]).astype(jnp.int32)
    ranks = jnp.cumsum(onehot, axis=0) - onehot
    rank = jnp.take_along_axis(ranks, bucket[:, None], axis=1)[:, 0]
    slot = bucket * CAP + rank

    dst_local = jnp.full((E_PAD,), -1, jnp.int32).at[slot].set(dst - bucket * G)
    src_pad = jnp.zeros((E_PAD,), jnp.int32).at[slot].set(src)
    e0_pad = jnp.zeros((E_PAD, 1), jnp.float32).at[slot].set(e)
    sn_pad = jnp.zeros((E_PAD, 1), jnp.float32).at[slot].set(snorm_e)

    hv = params["emb_h"][h]

    e_prev = e0_pad
    en_prev = e0_pad
    a_vec = params["emb_e"]["w"][0]
    c_vec = params["emb_e"]["b"]
    nlayers = len(params["layers"])
    for li, p in enumerate(params["layers"]):
        first = li == 0
        last = li == nlayers - 1
        w_ae = jnp.concatenate([p["A"]["w"], p["E"]["w"]], axis=1)
        b_ae = jnp.concatenate([p["A"]["b"], p["E"]["b"]], axis=0)
        w_db = jnp.concatenate([p["D"]["w"], p["B"]["w"]], axis=1)
        b_db = jnp.concatenate([p["D"]["b"], p["B"]["b"]], axis=0)
        ae = _linear(hv, w_ae, b_ae, block_m=400)
        Ah = ae[:, :HID]
        Eh = ae[:, HID:]
        Eh_pad = jnp.pad(Eh, ((0, N_PAD - N), (0, 0)))
        g1 = _take_sc(hv, src_pad)
        e_l, en_l, num_den, s12 = _edge_stage(
            e_prev, en_prev, a_vec, c_vec, g1, w_db, b_db, Eh_pad, dst_local,
            sn_pad, p["C"]["w"], p["C"]["b"], first, last,
        )
        num = num_den[:N, :HID]
        den = num_den[:N, HID:]
        h_new = Ah + num / (den + 1e-6)
        h_new = h_new * snorm_n
        h_new = _bn_nodes(h_new, p["bn_h_g"], p["bn_h_b"])
        hv = hv + jax.nn.relu(h_new)
        m_e = s12[0, :HID] / E_EDGES
        v_e = s12[0, HID:] / E_EDGES - m_e * m_e
        a_vec = p["bn_e_g"] / jnp.sqrt(v_e + 1e-5)
        c_vec = p["bn_e_b"] - m_e * a_vec
        e_prev, en_prev = e_l, en_l

    x = hv
    nmlp = len(params["mlp"])
    for i, lin in enumerate(params["mlp"]):
        x = _linear(x, lin["w"], lin["b"], block_m=2000)
        if i < nmlp - 1:
            x = jax.nn.relu(x)
    p_out = x
    hc = jnp.concatenate([hv, label], axis=1)
    w = jax.nn.sigmoid(_linear(hc, params["mlp2"]["w"], params["mlp2"]["b"], block_m=2000))
    w = jnp.tile(w, (1, label.shape[1]))
    w = jnp.clip(w, 0.0, jnp.asarray(delta, dtype=jnp.float32))
    ones = jnp.ones_like(label)
    max_entropy = jnp.full_like(label, 1.0 / label.shape[1])
    g_hat = (ones - w) * label + w * max_entropy
    return p_out, g_hat


# speedup vs baseline: 2.1937x; 1.0030x over previous
import jax
import jax.numpy as jnp
from jax import lax
from jax.experimental import pallas as pl
from jax.experimental.compute_on import compute_on

N = 10000
E_EDGES = 160000
HID = 256

G = 128
NBK = 79
CAP = 2560
E_PAD = NBK * CAP
N_PAD = NBK * G


def _linear_kernel(x_ref, w_ref, b_ref, o_ref):
    o_ref[...] = (
        jnp.dot(x_ref[...], w_ref[...], preferred_element_type=jnp.float32)
        + b_ref[...]
    )


def _linear(x, w, b, block_m=512):
    m, k = x.shape
    f = w.shape[1]
    grid = (pl.cdiv(m, block_m),)
    return pl.pallas_call(
        _linear_kernel,
        grid=grid,
        in_specs=[
            pl.BlockSpec((block_m, k), lambda i: (i, 0)),
            pl.BlockSpec((k, f), lambda i: (0, 0)),
            pl.BlockSpec((f,), lambda i: (0,)),
        ],
        out_specs=pl.BlockSpec((block_m, f), lambda i: (i, 0)),
        out_shape=jax.ShapeDtypeStruct((m, f), jnp.float32),
    )(x, w, b)


def _edge_kernel(
    eprev_ref, enprev_ref, a_ref, c_ref, g1_ref, wdb_ref, bdb_ref, eh_ref,
    dl_ref, sn_ref, cw_ref, cb_ref, el_ref, en_ref, nd_ref, s_ref, *, first, last
):
    if first:
        e_l = eprev_ref[...] * a_ref[...] + c_ref[...]
    else:
        e_l = eprev_ref[...].astype(jnp.float32) + jnp.maximum(
            enprev_ref[...].astype(jnp.float32) * a_ref[...] + c_ref[...], 0.0
        )
    if not last:
        el_ref[...] = e_l.astype(el_ref.dtype)
    ce = (
        jnp.dot(e_l.astype(jnp.bfloat16), cw_ref[...],
                preferred_element_type=jnp.float32)
        + cb_ref[...]
    )
    dl = dl_ref[0, 0, :]
    iota = lax.broadcasted_iota(jnp.int32, (CAP, G), 1)
    oh = (dl[:, None] == iota).astype(jnp.bfloat16)
    ehd = jnp.dot(oh, eh_ref[...], preferred_element_type=jnp.float32)
    db_s = (
        jnp.dot(g1_ref[...].astype(jnp.bfloat16), wdb_ref[...],
                preferred_element_type=jnp.float32)
        + bdb_ref[...]
    )
    e_pre = db_s[:, :HID] + ehd + ce
    sigma = jax.nn.sigmoid(e_pre)
    ps = jnp.concatenate(
        [sigma * db_s[:, HID:], sigma], axis=1
    ).astype(jnp.bfloat16)
    nd_ref[...] = lax.dot_general(
        oh, ps, (((0,), (0,)), ((), ())), preferred_element_type=jnp.float32
    )
    en = e_pre * sn_ref[...]
    if not last:
        en_ref[...] = en.astype(en_ref.dtype)
    bk = pl.program_id(0)

    @pl.when(bk == 0)
    def _():
        s_ref[...] = jnp.zeros_like(s_ref)

    s1 = jnp.sum(en, axis=0, keepdims=True)
    s2 = jnp.sum(en * en, axis=0, keepdims=True)
    s_ref[0:1, :] = s_ref[0:1, :] + jnp.concatenate([s1, s2], axis=1)


def _edge_stage(e_prev, en_prev, a, c, g1, wdb, bdb, eh, dst_local, sn, cw, cb,
                first, last):
    e_cols = 1 if first else HID
    kern = lambda *refs: _edge_kernel(*refs, first=first, last=last)
    out_shapes = [
        jax.ShapeDtypeStruct((E_PAD, HID), jnp.bfloat16),
        jax.ShapeDtypeStruct((E_PAD, HID), jnp.bfloat16),
        jax.ShapeDtypeStruct((N_PAD, 2 * HID), jnp.float32),
        jax.ShapeDtypeStruct((1, 2 * HID), jnp.float32),
    ]
    out_specs = [
        pl.BlockSpec((CAP, HID), lambda g: (g, 0)),
        pl.BlockSpec((CAP, HID), lambda g: (g, 0)),
        pl.BlockSpec((G, 2 * HID), lambda g: (g, 0)),
        pl.BlockSpec((1, 2 * HID), lambda g: (0, 0)),
    ]
    in_specs = [
        pl.BlockSpec((CAP, e_cols), lambda g: (g, 0)),
        pl.BlockSpec((CAP, e_cols), lambda g: (g, 0)),
        pl.BlockSpec((HID,), lambda g: (0,)),
        pl.BlockSpec((HID,), lambda g: (0,)),
        pl.BlockSpec((CAP, HID), lambda g: (g, 0)),
        pl.BlockSpec((HID, 2 * HID), lambda g: (0, 0)),
        pl.BlockSpec((2 * HID,), lambda g: (0,)),
        pl.BlockSpec((G, HID), lambda g: (g, 0)),
        pl.BlockSpec((1, 1, CAP), lambda g: (g, 0, 0)),
        pl.BlockSpec((CAP, 1), lambda g: (g, 0)),
        pl.BlockSpec((HID, HID), lambda g: (0, 0)),
        pl.BlockSpec((HID,), lambda g: (0,)),
    ]
    return pl.pallas_call(
        kern,
        grid=(NBK,),
        in_specs=in_specs,
        out_specs=out_specs,
        out_shape=out_shapes,
    )(e_prev, en_prev, a, c, g1, wdb, bdb, eh,
      dst_local.reshape(NBK, 1, CAP), sn, cw, cb)


@compute_on("tpu_sparsecore")
@jax.jit
def _take_sc(table, idx):
    return jnp.take(table, idx, axis=0, mode="clip")


def _bn_nodes(x, g, b):
    m = x.mean(axis=0)
    v = x.var(axis=0)
    return (x - m) / jnp.sqrt(v + 1e-5) * g + b


def kernel(h, e, edge_index, snorm_n, snorm_e, label, delta, params):
    src = edge_index[0]
    dst = edge_index[1]

    bucket = dst // G
    onehot = (bucket[:, None] == jnp.arange(NBK)[None, :]).astype(jnp.int32)
    ranks = jnp.cumsum(onehot, axis=0) - onehot
    rank = jnp.take_along_axis(ranks, bucket[:, None], axis=1)[:, 0]
    slot = bucket * CAP + rank

    dst_local = jnp.full((E_PAD,), -1, jnp.int32).at[slot].set(dst - bucket * G)
    src_pad = jnp.zeros((E_PAD,), jnp.int32).at[slot].set(src)
    e0_pad = jnp.zeros((E_PAD, 1), jnp.float32).at[slot].set(e)
    sn_pad = jnp.zeros((E_PAD, 1), jnp.float32).at[slot].set(snorm_e)

    hv = params["emb_h"][h]

    e_prev = e0_pad
    en_prev = e0_pad
    a_vec = params["emb_e"]["w"][0]
    c_vec = params["emb_e"]["b"]
    nlayers = len(params["layers"])
    for li, p in enumerate(params["layers"]):
        first = li == 0
        last = li == nlayers - 1
        w_ae = jnp.concatenate([p["A"]["w"], p["E"]["w"]], axis=1)
        b_ae = jnp.concatenate([p["A"]["b"], p["E"]["b"]], axis=0)
        w_db = jnp.concatenate([p["D"]["w"], p["B"]["w"]], axis=1)
        b_db = jnp.concatenate([p["D"]["b"], p["B"]["b"]], axis=0)
        ae = _linear(hv, w_ae, b_ae, block_m=400)
        Ah = ae[:, :HID]
        Eh = ae[:, HID:]
        Eh_pad = jnp.pad(Eh, ((0, N_PAD - N), (0, 0))).astype(jnp.bfloat16)
        g1 = _take_sc(hv, src_pad)
        e_l, en_l, num_den, s12 = _edge_stage(
            e_prev, en_prev, a_vec, c_vec, g1,
            w_db.astype(jnp.bfloat16), b_db, Eh_pad, dst_local,
            sn_pad, p["C"]["w"].astype(jnp.bfloat16), p["C"]["b"], first, last,
        )
        num = num_den[:N, :HID]
        den = num_den[:N, HID:]
        h_new = Ah + num / (den + 1e-6)
        h_new = h_new * snorm_n
        h_new = _bn_nodes(h_new, p["bn_h_g"], p["bn_h_b"])
        hv = hv + jax.nn.relu(h_new)
        m_e = s12[0, :HID] / E_EDGES
        v_e = s12[0, HID:] / E_EDGES - m_e * m_e
        a_vec = p["bn_e_g"] / jnp.sqrt(v_e + 1e-5)
        c_vec = p["bn_e_b"] - m_e * a_vec
        e_prev, en_prev = e_l, en_l

    x = hv
    nmlp = len(params["mlp"])
    for i, lin in enumerate(params["mlp"]):
        x = _linear(x, lin["w"], lin["b"], block_m=2000)
        if i < nmlp - 1:
            x = jax.nn.relu(x)
    p_out = x
    hc = jnp.concatenate([hv, label], axis=1)
    w = jax.nn.sigmoid(_linear(hc, params["mlp2"]["w"], params["mlp2"]["b"], block_m=2000))
    w = jnp.tile(w, (1, label.shape[1]))
    w = jnp.clip(w, 0.0, jnp.asarray(delta, dtype=jnp.float32))
    ones = jnp.ones_like(label)
    max_entropy = jnp.full_like(label, 1.0 / label.shape[1])
    g_hat = (ones - w) * label + w * max_entropy
    return p_out, g_hat
